# Initial kernel scaffold; baseline (speedup 1.0000x reference)
import jax
import jax.numpy as jnp
from jax.experimental import pallas as pl


def kernel(logt, xs, W_emb, b_emb, edge_w1, edge_b1, edge_w2, edge_b2, att_w, att_b, node_w1, node_b1, node_w2, node_b2, coord_w1, coord_b1, coord_w2):
    raise NotImplementedError("write your pallas kernel here")


# speedup vs baseline: 8.9151x; 8.9151x over previous
import numpy as np
import jax
import jax.numpy as jnp
from jax.experimental import pallas as pl

B, NP, ND = 256, 55, 3
HID, TEMB, NLAYERS = 64, 64, 3
DATA_SIGMA = 0.5
COORDS_RANGE = 15.0
NPP = NP * NP


def _selector_mats():
    i = np.repeat(np.arange(NP), NP)
    j = np.tile(np.arange(NP), NP)
    r_full = np.zeros((NPP, NP), np.float32)
    r_full[np.arange(NPP), i] = 1.0
    c_full = np.zeros((NPP, NP), np.float32)
    c_full[np.arange(NPP), j] = 1.0
    d_mat = r_full - c_full
    r0 = r_full.copy()
    r0[i == j] = 0.0
    return r0, c_full, d_mat


def _mm(a, b):
    return jax.lax.dot_general(a, b, (((1,), (0,)), ((), ())),
                               preferred_element_type=jnp.float32)


def _silu(v):
    return v * jax.nn.sigmoid(v)


def _egnn_body(logt_ref, xs_ref, r0_ref, c_ref, d_ref, r0t_ref,
               w_emb_ref, b_emb_ref,
               w1a_ref, w1b_ref, w1re_ref, b1_ref, w2_ref, b2_ref,
               attw_ref, attb_ref,
               nw1h_ref, nw1a_ref, nb1_ref, nw2_ref, nb2_ref,
               cw1_ref, cb1_ref, cw2_ref,
               out_ref):
    lt = logt_ref[0, 0, 0]
    t = jnp.exp(lt)
    denom = DATA_SIGMA * DATA_SIGMA + t * t
    c_in = 1.0 / jnp.sqrt(denom)
    c_skip = (DATA_SIGMA * DATA_SIGMA) / denom
    c_out = DATA_SIGMA * t / jnp.sqrt(denom)

    half = TEMB // 2
    fidx = jax.lax.broadcasted_iota(jnp.int32, (1, half), 1).astype(jnp.float32)
    freqs = jnp.exp(fidx * (np.float32(np.log(1.0 / 10000.0) / half)))
    ang = (lt / 4.0) * freqs
    temb = jnp.concatenate([jnp.cos(ang), jnp.sin(ang)], axis=1)
    h0 = _mm(temb, w_emb_ref[...]) + b_emb_ref[...]
    h = jnp.broadcast_to(h0, (NP, HID))

    x_in = xs_ref[0]
    x0 = x_in * c_in
    x = x0

    r0 = r0_ref[...]
    c_sel = c_ref[...]
    d_sel = d_ref[...]
    r0t = r0t_ref[...]

    d0 = _mm(d_sel, x0)
    e_attr = jnp.sum(d0 * d0, axis=1, keepdims=True)

    for i in range(NLAYERS):
        diff = _mm(d_sel, x)
        radial = jnp.sum(diff * diff, axis=1, keepdims=True)
        cdiff = diff / (jnp.sqrt(radial + 1e-8) + 1.0)

        a_rows = _mm(h, w1a_ref[i])
        b_rows = _mm(h, w1b_ref[i])
        pre1 = (_mm(r0, a_rows) + _mm(c_sel, b_rows)
                + radial * w1re_ref[i, 0:1] + e_attr * w1re_ref[i, 1:2]
                + b1_ref[i])
        m1 = _silu(pre1)
        m2 = _silu(_mm(m1, w2_ref[i]) + b2_ref[i])
        att = jax.nn.sigmoid(
            jnp.sum(m2 * attw_ref[i], axis=1, keepdims=True) + attb_ref[i])
        m = m2 * att

        cm = _silu(_mm(m, cw1_ref[i]) + cb1_ref[i])
        cw = jnp.tanh(jnp.sum(cm * cw2_ref[i], axis=1, keepdims=True))
        trans = cdiff * (cw * COORDS_RANGE)

        x = x + _mm(r0t, trans)
        agg = _mm(r0t, m)

        n1 = _silu(_mm(h, nw1h_ref[i]) + _mm(agg, nw1a_ref[i]) + nb1_ref[i])
        h = h + _mm(n1, nw2_ref[i]) + nb2_ref[i]

    vel = x - x0
    out_ref[0] = x_in * c_skip + vel * c_out


def kernel(logt, xs, W_emb, b_emb, edge_w1, edge_b1, edge_w2, edge_b2,
           att_w, att_b, node_w1, node_b1, node_w2, node_b2,
           coord_w1, coord_b1, coord_w2):
    r0_np, c_np, d_np = _selector_mats()
    r0 = jnp.asarray(r0_np)
    c_sel = jnp.asarray(c_np)
    d_sel = jnp.asarray(d_np)
    r0t = jnp.asarray(np.ascontiguousarray(r0_np.T))

    logt2 = logt[:, None, None]
    xs3 = xs.reshape(B, NP, ND)

    w1a = edge_w1[:, :HID, :]
    w1b = edge_w1[:, HID:2 * HID, :]
    w1re = edge_w1[:, 2 * HID:, :]
    b1 = edge_b1[:, None, :]
    b2 = edge_b2[:, None, :]
    attw = jnp.transpose(att_w, (0, 2, 1))
    attb = att_b[:, None, :]
    nw1h = node_w1[:, :HID, :]
    nw1a = node_w1[:, HID:, :]
    nb1 = node_b1[:, None, :]
    nb2 = node_b2[:, None, :]
    cb1 = coord_b1[:, None, :]
    cw2 = jnp.transpose(coord_w2, (0, 2, 1))
    b_emb2 = b_emb[None, :]

    grid = (B,)
    in_specs = [
        pl.BlockSpec((1, 1, 1), lambda s: (s, 0, 0)),
        pl.BlockSpec((1, NP, ND), lambda s: (s, 0, 0)),
        pl.BlockSpec((NPP, NP), lambda s: (0, 0)),
        pl.BlockSpec((NPP, NP), lambda s: (0, 0)),
        pl.BlockSpec((NPP, NP), lambda s: (0, 0)),
        pl.BlockSpec((NP, NPP), lambda s: (0, 0)),
        pl.BlockSpec((TEMB, HID), lambda s: (0, 0)),
        pl.BlockSpec((1, HID), lambda s: (0, 0)),
        pl.BlockSpec((NLAYERS, HID, HID), lambda s: (0, 0, 0)),
        pl.BlockSpec((NLAYERS, HID, HID), lambda s: (0, 0, 0)),
        pl.BlockSpec((NLAYERS, 2, HID), lambda s: (0, 0, 0)),
        pl.BlockSpec((NLAYERS, 1, HID), lambda s: (0, 0, 0)),
        pl.BlockSpec((NLAYERS, HID, HID), lambda s: (0, 0, 0)),
        pl.BlockSpec((NLAYERS, 1, HID), lambda s: (0, 0, 0)),
        pl.BlockSpec((NLAYERS, 1, HID), lambda s: (0, 0, 0)),
        pl.BlockSpec((NLAYERS, 1, 1), lambda s: (0, 0, 0)),
        pl.BlockSpec((NLAYERS, HID, HID), lambda s: (0, 0, 0)),
        pl.BlockSpec((NLAYERS, HID, HID), lambda s: (0, 0, 0)),
        pl.BlockSpec((NLAYERS, 1, HID), lambda s: (0, 0, 0)),
        pl.BlockSpec((NLAYERS, HID, HID), lambda s: (0, 0, 0)),
        pl.BlockSpec((NLAYERS, 1, HID), lambda s: (0, 0, 0)),
        pl.BlockSpec((NLAYERS, HID, HID), lambda s: (0, 0, 0)),
        pl.BlockSpec((NLAYERS, 1, HID), lambda s: (0, 0, 0)),
        pl.BlockSpec((NLAYERS, 1, HID), lambda s: (0, 0, 0)),
    ]
    out3 = pl.pallas_call(
        _egnn_body,
        grid=grid,
        in_specs=in_specs,
        out_specs=pl.BlockSpec((1, NP, ND), lambda s: (s, 0, 0)),
        out_shape=jax.ShapeDtypeStruct((B, NP, ND), jnp.float32),
    )(logt2, xs3, r0, c_sel, d_sel, r0t, W_emb, b_emb2,
      w1a, w1b, w1re, b1, edge_w2, b2, attw, attb,
      nw1h, nw1a, nb1, node_w2, nb2, coord_w1, cb1, cw2)
    return out3.reshape(B, NP * ND)
